# baseline (speedup 1.0000x reference)
import functools

import jax
import jax.numpy as jnp
from jax import lax
from jax.experimental import pallas as pl
from jax.experimental.pallas import tpu as pltpu
from jax.experimental.pallas import tpu_sc as plsc

B = 16384
SL = 73
DC, DR, DB = 16, 8, 6

_info = plsc.get_sparse_core_info()
_NC, _NS, _L = 1, _info.num_subcores, _info.num_lanes
NW = _NC * _NS
BPW = B // NW
CHUNK = 128
NCH = BPW // CHUNK


def _sc_body(ic_hbm, ir_hbm, ib_hbm, wc_hbm, wr_hbm, wb_hbm,
             oc_hbm, orr_hbm, ob_hbm,
             idxc_v, idxr_v, idxb_v,
             rc_v, rr_v, rb_v,
             wc_t, wr_t, wb_t, wc_s, wr_s, wb_s,
             sem_e, sem_g, sem_o):
    sid = lax.axis_index("s")
    wid = sid * _NC + lax.axis_index("c")
    base = wid * BPW

    i1 = pltpu.async_copy(ic_hbm.at[pl.ds(base, BPW)], idxc_v, sem_e)
    i2 = pltpu.async_copy(ir_hbm.at[pl.ds(base, BPW)], idxr_v, sem_e)
    i3 = pltpu.async_copy(ib_hbm.at[pl.ds(base, BPW)], idxb_v, sem_e)

    @pl.when(sid == 0)
    def _stage_tables():
        pltpu.sync_copy(wc_hbm, wc_t)
        pltpu.sync_copy(wr_hbm, wr_t)
        pltpu.sync_copy(wb_hbm, wb_t)
        pltpu.sync_copy(wc_t, wc_s)
        pltpu.sync_copy(wr_t, wr_s)
        pltpu.sync_copy(wb_t, wb_s)

    plsc.subcore_barrier()
    i1.wait()
    i2.wait()
    i3.wait()

    g1 = pltpu.async_copy(wc_s.at[idxc_v], rc_v, sem_g)
    g2 = pltpu.async_copy(wr_s.at[idxr_v], rr_v, sem_g)
    g3 = pltpu.async_copy(wb_s.at[idxb_v], rb_v, sem_g)
    g1.wait()
    g2.wait()
    g3.wait()

    o1 = pltpu.async_copy(rc_v, oc_hbm.at[pl.ds(base, BPW)], sem_o)
    o2 = pltpu.async_copy(rr_v, orr_hbm.at[pl.ds(base, BPW)], sem_o)
    o3 = pltpu.async_copy(rb_v, ob_hbm.at[pl.ds(base, BPW)], sem_o)
    o1.wait()
    o2.wait()
    o3.wait()


_sc_call = functools.partial(
    pl.kernel,
    mesh=plsc.VectorSubcoreMesh(core_axis_name="c", subcore_axis_name="s",
                                num_cores=_NC),
    compiler_params=pltpu.CompilerParams(use_tc_tiling_on_sc=False),
    out_type=(
        jax.ShapeDtypeStruct((B, DC), jnp.float32),
        jax.ShapeDtypeStruct((B, DR), jnp.float32),
        jax.ShapeDtypeStruct((B, DR), jnp.float32),
    ),
    scratch_types=[
        pltpu.VMEM((BPW,), jnp.int32),
        pltpu.VMEM((BPW,), jnp.int32),
        pltpu.VMEM((BPW,), jnp.int32),
        pltpu.VMEM((BPW, DC), jnp.float32),
        pltpu.VMEM((BPW, DR), jnp.float32),
        pltpu.VMEM((BPW, DR), jnp.float32),
        pltpu.VMEM((101, DC), jnp.float32),
        pltpu.VMEM((9, DR), jnp.float32),
        pltpu.VMEM((51, DR), jnp.float32),
        pltpu.VMEM_SHARED((101, DC), jnp.float32),
        pltpu.VMEM_SHARED((9, DR), jnp.float32),
        pltpu.VMEM_SHARED((51, DR), jnp.float32),
        pltpu.SemaphoreType.DMA,
        pltpu.SemaphoreType.DMA,
        pltpu.SemaphoreType.DMA,
    ],
)(_sc_body)


_RB = 2048


def _ids_body(x_ref, ic_ref, ir_ref, ib_ref):
    blk = x_ref[...]
    ic_ref[...] = blk[:, 0].astype(jnp.int32)
    ir_ref[...] = blk[:, 1].astype(jnp.int32)
    ib_ref[...] = blk[:, 2].astype(jnp.int32)


_ids_call = pl.pallas_call(
    _ids_body,
    grid=(B // _RB,),
    in_specs=[pl.BlockSpec((_RB, 8), lambda i: (i, 0))],
    out_specs=(
        pl.BlockSpec((_RB,), lambda i: (i,)),
        pl.BlockSpec((_RB,), lambda i: (i,)),
        pl.BlockSpec((_RB,), lambda i: (i,)),
    ),
    out_shape=(
        jax.ShapeDtypeStruct((B,), jnp.int32),
        jax.ShapeDtypeStruct((B,), jnp.int32),
        jax.ShapeDtypeStruct((B,), jnp.int32),
    ),
)


def _states_body(x_ref, os_ref):
    os_ref[...] = x_ref[:, 3:]


_states_call = pl.pallas_call(
    _states_body,
    grid=(B // _RB,),
    in_specs=[pl.BlockSpec((_RB, SL), lambda i: (i, 0))],
    out_specs=pl.BlockSpec((_RB, SL - 3), lambda i: (i, 0)),
    out_shape=jax.ShapeDtypeStruct((B, SL - 3), jnp.float32),
)


def kernel(x, W_char, W_role, W_buff):
    wb8 = jnp.pad(W_buff, ((0, 0), (0, DR - DB)))
    ic, ir, ib = _ids_call(x[:, :8])
    oc, orr, ob8 = _sc_call(ic, ir, ib, W_char, W_role, wb8)
    os = _states_call(x)
    return oc, orr, ob8[:, :DB], os


# speedup vs baseline: 1.0579x; 1.0579x over previous
import functools

import jax
import jax.numpy as jnp
from jax import lax
from jax.experimental import pallas as pl
from jax.experimental.pallas import tpu as pltpu
from jax.experimental.pallas import tpu_sc as plsc

B = 16384
SL = 73
DC, DR, DB = 16, 8, 6

_info = plsc.get_sparse_core_info()
_NC, _NS, _L = 1, _info.num_subcores, _info.num_lanes
NW = _NC * _NS
BPW = B // NW
FPW = BPW * SL
NG = BPW // _L


def _sc_body(xf_hbm, wc_hbm, wr_hbm, wb_hbm,
             oc_hbm, orr_hbm, ob_hbm,
             colc_v, colr_v, colb_v,
             idxc_v, idxr_v, idxb_v,
             rc_v, rr_v, rb_v,
             wc_t, wr_t, wb_t, wc_s, wr_s, wb_s,
             sem_e, sem_g, sem_o):
    sid = lax.axis_index("s")
    wid = sid * _NC + lax.axis_index("c")
    base = wid * BPW

    @pl.when(sid == 0)
    def _stage_tables():
        pltpu.sync_copy(wc_hbm, wc_t)
        pltpu.sync_copy(wr_hbm, wr_t)
        pltpu.sync_copy(wb_hbm, wb_t)
        pltpu.sync_copy(wc_t, wc_s)
        pltpu.sync_copy(wr_t, wr_s)
        pltpu.sync_copy(wb_t, wb_s)

    lanes = lax.iota(jnp.int32, _L)
    for g in range(NG):
        s = pl.ds(g * _L, _L)
        flat0 = (base + g * _L + lanes) * SL
        idxc_v[s] = flat0
        idxr_v[s] = flat0 + 1
        idxb_v[s] = flat0 + 2

    e1 = pltpu.async_copy(xf_hbm.at[idxc_v], colc_v, sem_e)
    e2 = pltpu.async_copy(xf_hbm.at[idxr_v], colr_v, sem_e)
    e3 = pltpu.async_copy(xf_hbm.at[idxb_v], colb_v, sem_e)
    e1.wait()
    e2.wait()
    e3.wait()

    for g in range(NG):
        s = pl.ds(g * _L, _L)
        idxc_v[s] = colc_v[s].astype(jnp.int32)
        idxr_v[s] = colr_v[s].astype(jnp.int32)
        idxb_v[s] = colb_v[s].astype(jnp.int32)

    plsc.subcore_barrier()

    g1 = pltpu.async_copy(wc_s.at[idxc_v], rc_v, sem_g)
    g2 = pltpu.async_copy(wr_s.at[idxr_v], rr_v, sem_g)
    g3 = pltpu.async_copy(wb_s.at[idxb_v], rb_v, sem_g)
    g1.wait()
    g2.wait()
    g3.wait()

    o1 = pltpu.async_copy(rc_v, oc_hbm.at[pl.ds(base, BPW)], sem_o)
    o2 = pltpu.async_copy(rr_v, orr_hbm.at[pl.ds(base, BPW)], sem_o)
    o3 = pltpu.async_copy(rb_v, ob_hbm.at[pl.ds(base, BPW)], sem_o)
    o1.wait()
    o2.wait()
    o3.wait()


_sc_call = functools.partial(
    pl.kernel,
    mesh=plsc.VectorSubcoreMesh(core_axis_name="c", subcore_axis_name="s",
                                num_cores=_NC),
    compiler_params=pltpu.CompilerParams(use_tc_tiling_on_sc=False),
    out_type=(
        jax.ShapeDtypeStruct((B, DC), jnp.float32),
        jax.ShapeDtypeStruct((B, DR), jnp.float32),
        jax.ShapeDtypeStruct((B, DR), jnp.float32),
    ),
    scratch_types=[
        pltpu.VMEM((BPW,), jnp.float32),
        pltpu.VMEM((BPW,), jnp.float32),
        pltpu.VMEM((BPW,), jnp.float32),
        pltpu.VMEM((BPW,), jnp.int32),
        pltpu.VMEM((BPW,), jnp.int32),
        pltpu.VMEM((BPW,), jnp.int32),
        pltpu.VMEM((BPW, DC), jnp.float32),
        pltpu.VMEM((BPW, DR), jnp.float32),
        pltpu.VMEM((BPW, DR), jnp.float32),
        pltpu.VMEM((101, DC), jnp.float32),
        pltpu.VMEM((9, DR), jnp.float32),
        pltpu.VMEM((51, DR), jnp.float32),
        pltpu.VMEM_SHARED((101, DC), jnp.float32),
        pltpu.VMEM_SHARED((9, DR), jnp.float32),
        pltpu.VMEM_SHARED((51, DR), jnp.float32),
        pltpu.SemaphoreType.DMA,
        pltpu.SemaphoreType.DMA,
        pltpu.SemaphoreType.DMA,
    ],
)(_sc_body)


_RB = 2048


def _states_body(x_ref, os_ref):
    os_ref[...] = x_ref[:, 3:]


_states_call = pl.pallas_call(
    _states_body,
    grid=(B // _RB,),
    in_specs=[pl.BlockSpec((_RB, SL), lambda i: (i, 0))],
    out_specs=pl.BlockSpec((_RB, SL - 3), lambda i: (i, 0)),
    out_shape=jax.ShapeDtypeStruct((B, SL - 3), jnp.float32),
)


def kernel(x, W_char, W_role, W_buff):
    wb8 = jnp.pad(W_buff, ((0, 0), (0, DR - DB)))
    oc, orr, ob8 = _sc_call(x.reshape(-1), W_char, W_role, wb8)
    os = _states_call(x)
    return oc, orr, ob8[:, :DB], os


# speedup vs baseline: 1.2419x; 1.1740x over previous
import functools

import jax
import jax.numpy as jnp
from jax import lax
from jax.experimental import pallas as pl
from jax.experimental.pallas import tpu as pltpu
from jax.experimental.pallas import tpu_sc as plsc

B = 16384
SL = 73
DC, DR, DB = 16, 8, 6
VC, VR, VB = 101, 9, 51

_info = plsc.get_sparse_core_info()
_NC, _NS, _L = 1, _info.num_subcores, _info.num_lanes
NW = _NC * _NS
BPW = B // NW


def _sc_body(ic_hbm, wc_hbm, oc_hbm,
             idxc_v, rc_v, wc_t, wc_s, sem_e, sem_g, sem_o):
    sid = lax.axis_index("s")
    wid = sid * _NC + lax.axis_index("c")
    base = wid * BPW

    i1 = pltpu.async_copy(ic_hbm.at[pl.ds(base, BPW)], idxc_v, sem_e)

    @pl.when(sid == 0)
    def _stage_table():
        pltpu.sync_copy(wc_hbm, wc_t)
        pltpu.sync_copy(wc_t, wc_s)

    plsc.subcore_barrier()
    i1.wait()

    pltpu.async_copy(wc_s.at[idxc_v], rc_v, sem_g).wait()

    pltpu.async_copy(rc_v, oc_hbm.at[pl.ds(base, BPW)], sem_o).wait()


_sc_call = functools.partial(
    pl.kernel,
    mesh=plsc.VectorSubcoreMesh(core_axis_name="c", subcore_axis_name="s",
                                num_cores=_NC),
    compiler_params=pltpu.CompilerParams(use_tc_tiling_on_sc=False),
    out_type=jax.ShapeDtypeStruct((B, DC), jnp.float32),
    scratch_types=[
        pltpu.VMEM((BPW,), jnp.int32),
        pltpu.VMEM((BPW, DC), jnp.float32),
        pltpu.VMEM((VC, DC), jnp.float32),
        pltpu.VMEM_SHARED((VC, DC), jnp.float32),
        pltpu.SemaphoreType.DMA,
        pltpu.SemaphoreType.DMA,
        pltpu.SemaphoreType.DMA,
    ],
)(_sc_body)


_RB = 2048


def _tc_body(x_ref, wr_ref, wb_ref, ic_ref, os_ref, orr_ref, ob_ref):
    blk = x_ref[...]
    ic_ref[...] = blk[:, 0].astype(jnp.int32)
    os_ref[...] = blk[:, 3:]
    role = blk[:, 1:2].astype(jnp.int32)
    oh_r = (role == lax.broadcasted_iota(jnp.int32, (_RB, VR), 1))
    orr_ref[...] = jnp.dot(oh_r.astype(jnp.float32), wr_ref[...],
                           preferred_element_type=jnp.float32)
    buff = blk[:, 2:3].astype(jnp.int32)
    oh_b = (buff == lax.broadcasted_iota(jnp.int32, (_RB, VB), 1))
    ob_ref[...] = jnp.dot(oh_b.astype(jnp.float32), wb_ref[...],
                          preferred_element_type=jnp.float32)


_tc_call = pl.pallas_call(
    _tc_body,
    grid=(B // _RB,),
    in_specs=[
        pl.BlockSpec((_RB, SL), lambda i: (i, 0)),
        pl.BlockSpec((VR, DR), lambda i: (0, 0)),
        pl.BlockSpec((VB, DB), lambda i: (0, 0)),
    ],
    out_specs=(
        pl.BlockSpec((_RB,), lambda i: (i,)),
        pl.BlockSpec((_RB, SL - 3), lambda i: (i, 0)),
        pl.BlockSpec((_RB, DR), lambda i: (i, 0)),
        pl.BlockSpec((_RB, DB), lambda i: (i, 0)),
    ),
    out_shape=(
        jax.ShapeDtypeStruct((B,), jnp.int32),
        jax.ShapeDtypeStruct((B, SL - 3), jnp.float32),
        jax.ShapeDtypeStruct((B, DR), jnp.float32),
        jax.ShapeDtypeStruct((B, DB), jnp.float32),
    ),
)


def kernel(x, W_char, W_role, W_buff):
    ic, os, orr, ob = _tc_call(x, W_role, W_buff)
    oc = _sc_call(ic, W_char)
    return oc, orr, ob, os
